# baseline (speedup 1.0000x reference)
import functools
import math

import jax
import jax.numpy as jnp
from jax.experimental import pallas as pl


_K_VALUE = 2
_CAPACITY_FACTOR = 1.25
_MIN_CAPACITY = 4


def _capacity(num_tokens, num_experts):
    cap = math.floor(_K_VALUE * _CAPACITY_FACTOR * num_tokens / num_experts)
    cap += cap % 2
    return max(cap, _MIN_CAPACITY)


def _cumsum0(x):
    n = x.shape[0]
    k = 1
    while k < n:
        shifted = jnp.concatenate(
            [jnp.zeros((k, x.shape[1]), x.dtype), x[:-k, :]], axis=0)
        x = x + shifted
        k *= 2
    return x


def _routing_kernel(x_ref, meta_i_ref, meta_f_ref, used_ref, *, cap):
    x = x_ref[:, :]
    nt, ne = x.shape
    xmax = jnp.max(x, axis=1, keepdims=True)
    unnorm = jnp.exp(x - xmax)
    probs = unnorm / jnp.sum(unnorm, axis=1, keepdims=True)

    lane = jax.lax.broadcasted_iota(jnp.int32, (nt, ne), 1)
    pmax = jnp.max(probs, axis=1, keepdims=True)
    e1 = jnp.min(jnp.where(probs == pmax, lane, ne), axis=1, keepdims=True)
    mask1 = lane == e1
    probs2 = jnp.where(mask1, -jnp.inf, probs)
    pmax2 = jnp.max(probs2, axis=1, keepdims=True)
    e2 = jnp.min(jnp.where(probs2 == pmax2, lane, ne), axis=1, keepdims=True)
    mask2 = lane == e2

    c1 = _cumsum0(mask1.astype(jnp.int32))
    c2 = _cumsum0(mask2.astype(jnp.int32))
    tot1 = c1[nt - 1:nt, :]
    rank1 = c1 - 1
    rank2 = c2 - 1 + tot1

    keep1 = mask1 & (rank1 < cap)
    keep2 = mask2 & (rank2 < cap)
    used_ref[:, :] = jnp.sum(
        keep1.astype(jnp.int32) + keep2.astype(jnp.int32), axis=0,
        keepdims=True)

    r1tok = jnp.sum(jnp.where(keep1, rank1, 0), axis=1, keepdims=True)
    r2tok = jnp.sum(jnp.where(keep2, rank2, 0), axis=1, keepdims=True)
    k1tok = jnp.any(keep1, axis=1, keepdims=True)
    k2tok = jnp.any(keep2, axis=1, keepdims=True)
    p1 = jnp.where(k1tok, e1 * cap + r1tok, -1)
    p2 = jnp.where(k2tok, e2 * cap + r2tok, -1)
    w1 = jnp.sum(jnp.where(keep1, probs, 0.0), axis=1, keepdims=True)
    w2 = jnp.sum(jnp.where(keep2, probs, 0.0), axis=1, keepdims=True)

    meta_i_ref[:, :] = jnp.where(lane == 0, p1, jnp.where(lane == 1, p2, 0))
    meta_f_ref[:, :] = jnp.where(lane == 0, w1, jnp.where(lane == 1, w2, 0.0))


def _writer_kernel(meta_i_ref, meta_f_ref, cb_ref, sec_ref, *, cap):
    t, ne = meta_i_ref.shape
    p1 = meta_i_ref[:, 0:1].reshape(t, 1, 1)
    p2 = meta_i_ref[:, 1:2].reshape(t, 1, 1)
    w1 = meta_f_ref[:, 0:1].reshape(t, 1, 1)
    w2 = meta_f_ref[:, 1:2].reshape(t, 1, 1)
    pos = (jax.lax.broadcasted_iota(jnp.int32, (t, ne, cap), 1) * cap
           + jax.lax.broadcasted_iota(jnp.int32, (t, ne, cap), 2))
    hit1 = pos == p1
    hit2 = pos == p2
    cb_ref[:, :, :] = jnp.where(hit1, w1, jnp.where(hit2, w2, 0.0))
    sec_ref[:, :, :] = hit1 | hit2


def kernel(inputs):
    nt, ne = inputs.shape
    cap = _capacity(nt, ne)
    plane = ne * cap

    meta_i, meta_f, used = pl.pallas_call(
        functools.partial(_routing_kernel, cap=cap),
        out_shape=[
            jax.ShapeDtypeStruct((nt, ne), jnp.int32),
            jax.ShapeDtypeStruct((nt, ne), jnp.float32),
            jax.ShapeDtypeStruct((1, ne), jnp.int32),
        ],
    )(inputs)

    blk = 256
    cb_weight, sec_mask = pl.pallas_call(
        functools.partial(_writer_kernel, cap=cap),
        grid=(nt // blk,),
        in_specs=[
            pl.BlockSpec((blk, ne), lambda i: (i, 0)),
            pl.BlockSpec((blk, ne), lambda i: (i, 0)),
        ],
        out_specs=[
            pl.BlockSpec((blk, ne, cap), lambda i: (i, 0, 0)),
            pl.BlockSpec((blk, ne, cap), lambda i: (i, 0, 0)),
        ],
        out_shape=[
            jax.ShapeDtypeStruct((nt, ne, cap), jnp.float32),
            jax.ShapeDtypeStruct((nt, ne, cap), jnp.bool_),
        ],
    )(meta_i, meta_f)

    used_capacity = used.reshape(ne)
    return (used_capacity, cb_weight, sec_mask)


# speedup vs baseline: 1.0226x; 1.0226x over previous
import functools
import math

import jax
import jax.numpy as jnp
from jax.experimental import pallas as pl
from jax.experimental.pallas import tpu as pltpu


_K_VALUE = 2
_CAPACITY_FACTOR = 1.25
_MIN_CAPACITY = 4


def _capacity(num_tokens, num_experts):
    cap = math.floor(_K_VALUE * _CAPACITY_FACTOR * num_tokens / num_experts)
    cap += cap % 2
    return max(cap, _MIN_CAPACITY)


def _cumsum0(x):
    n = x.shape[0]
    k = 1
    while k < n:
        shifted = jnp.concatenate(
            [jnp.zeros((k, x.shape[1]), x.dtype), x[:-k, :]], axis=0)
        x = x + shifted
        k *= 2
    return x


def _routing(x, cap):
    nt, ne = x.shape
    xmax = jnp.max(x, axis=1, keepdims=True)
    unnorm = jnp.exp(x - xmax)
    probs = unnorm / jnp.sum(unnorm, axis=1, keepdims=True)

    lane = jax.lax.broadcasted_iota(jnp.int32, (nt, ne), 1)
    pmax = jnp.max(probs, axis=1, keepdims=True)
    e1 = jnp.min(jnp.where(probs == pmax, lane, ne), axis=1, keepdims=True)
    mask1 = lane == e1
    probs2 = jnp.where(mask1, -jnp.inf, probs)
    pmax2 = jnp.max(probs2, axis=1, keepdims=True)
    e2 = jnp.min(jnp.where(probs2 == pmax2, lane, ne), axis=1, keepdims=True)
    mask2 = lane == e2

    c1 = _cumsum0(mask1.astype(jnp.int32))
    c2 = _cumsum0(mask2.astype(jnp.int32))
    tot1 = c1[nt - 1:nt, :]
    rank1 = c1 - 1
    rank2 = c2 - 1 + tot1

    keep1 = mask1 & (rank1 < cap)
    keep2 = mask2 & (rank2 < cap)
    used = jnp.sum(
        keep1.astype(jnp.int32) + keep2.astype(jnp.int32), axis=0,
        keepdims=True)

    r1tok = jnp.sum(jnp.where(keep1, rank1, 0), axis=1, keepdims=True)
    r2tok = jnp.sum(jnp.where(keep2, rank2, 0), axis=1, keepdims=True)
    k1tok = jnp.any(keep1, axis=1, keepdims=True)
    k2tok = jnp.any(keep2, axis=1, keepdims=True)
    p1 = jnp.where(k1tok, e1 * cap + r1tok, -1)
    p2 = jnp.where(k2tok, e2 * cap + r2tok, -1)
    w1 = jnp.sum(jnp.where(keep1, probs, 0.0), axis=1, keepdims=True)
    w2 = jnp.sum(jnp.where(keep2, probs, 0.0), axis=1, keepdims=True)

    meta_i = jnp.where(lane == 0, p1, jnp.where(lane == 1, p2, 0))
    meta_f = jnp.where(lane == 0, w1, jnp.where(lane == 1, w2, 0.0))
    return meta_i, meta_f, used


def _fused_kernel(x_ref, cb_ref, sec_ref, used_ref, meta_i_s, meta_f_s, *,
                  cap, blk):
    i = pl.program_id(0)

    @pl.when(i == 0)
    def _():
        meta_i, meta_f, used = _routing(x_ref[:, :], cap)
        meta_i_s[:, :] = meta_i
        meta_f_s[:, :] = meta_f
        used_ref[:, :] = used

    ne = x_ref.shape[1]
    base = i * blk
    p1 = meta_i_s[pl.ds(base, blk), 0:1].reshape(blk, 1, 1)
    p2 = meta_i_s[pl.ds(base, blk), 1:2].reshape(blk, 1, 1)
    w1 = meta_f_s[pl.ds(base, blk), 0:1].reshape(blk, 1, 1)
    w2 = meta_f_s[pl.ds(base, blk), 1:2].reshape(blk, 1, 1)
    pos = (jax.lax.broadcasted_iota(jnp.int32, (blk, ne, cap), 1) * cap
           + jax.lax.broadcasted_iota(jnp.int32, (blk, ne, cap), 2))
    hit1 = pos == p1
    hit2 = pos == p2
    cb_ref[:, :, :] = jnp.where(hit1, w1, jnp.where(hit2, w2, 0.0))
    sec_ref[:, :, :] = hit1 | hit2


def kernel(inputs):
    nt, ne = inputs.shape
    cap = _capacity(nt, ne)
    blk = 128

    cb_weight, sec_mask, used = pl.pallas_call(
        functools.partial(_fused_kernel, cap=cap, blk=blk),
        grid=(nt // blk,),
        in_specs=[pl.BlockSpec((nt, ne), lambda i: (0, 0))],
        out_specs=[
            pl.BlockSpec((blk, ne, cap), lambda i: (i, 0, 0)),
            pl.BlockSpec((blk, ne, cap), lambda i: (i, 0, 0)),
            pl.BlockSpec((1, ne), lambda i: (0, 0)),
        ],
        out_shape=[
            jax.ShapeDtypeStruct((nt, ne, cap), jnp.float32),
            jax.ShapeDtypeStruct((nt, ne, cap), jnp.bool_),
            jax.ShapeDtypeStruct((1, ne), jnp.int32),
        ],
        scratch_shapes=[
            pltpu.VMEM((nt, ne), jnp.int32),
            pltpu.VMEM((nt, ne), jnp.float32),
        ],
    )(inputs)

    return (used.reshape(ne), cb_weight, sec_mask)
